# Initial kernel scaffold; baseline (speedup 1.0000x reference)
import jax
import jax.numpy as jnp
from jax.experimental import pallas as pl


def kernel(x, W_mu1, a_s_mu1, a_d_mu1, W_ls1, a_s_ls1, a_d_ls1, W_mu2, a_s_mu2, a_d_mu2, W_ls2, a_s_ls2, a_d_ls2, edge_index):
    raise NotImplementedError("write your pallas kernel here")


# speedup vs baseline: 63.6299x; 63.6299x over previous
import functools

import jax
import jax.numpy as jnp
from jax import lax
from jax.experimental import pallas as pl
from jax.experimental.pallas import tpu as pltpu
from jax.experimental.pallas import tpu_sc as plsc

N = 10000
E = 320000
D_IN = 128
LAT = 32
MAX_LOGSTD = 10.0

HS_W = 144
AD_W = 16
NC = 2
NS = 16
NW = NC * NS
CHUNK = 80
N_CHUNKS = E // CHUNK
CHUNKS_PER_TILE = N_CHUNKS // NW
N_PAD = 10240
ROWS_PER_TILE = N_PAD // NS

_ROWBLK = 1000
_FROWBLK = 1024


def _proj_body(x_ref, w_ref, amat_s_ref, amat_d_ref, hs_ref, ad_ref):
    h = jnp.dot(x_ref[...], w_ref[...], preferred_element_type=jnp.float32)
    hs_ref[:, 0:D_IN] = h
    hs_ref[:, D_IN:HS_W] = jnp.dot(h, amat_s_ref[...],
                                   preferred_element_type=jnp.float32)
    ad_ref[...] = jnp.dot(h, amat_d_ref[...],
                          preferred_element_type=jnp.float32)


def _finalize_body(p_ref, noise_ref, z_ref):
    a = p_ref[0] + p_ref[1]
    eps = 1e-16
    mu1 = a[:, 0:32] / (a[:, 128:129] + eps)
    ls1 = a[:, 32:64] / (a[:, 129:130] + eps)
    mu2 = a[:, 64:96] / (a[:, 130:131] + eps)
    ls2 = a[:, 96:128] / (a[:, 131:132] + eps)
    z_ref[:, 0:32] = mu1 + noise_ref[:, 0:32] * jnp.exp(
        jnp.minimum(ls1, MAX_LOGSTD))
    z_ref[:, 32:64] = mu2 + noise_ref[:, 32:64] * jnp.exp(
        jnp.minimum(ls2, MAX_LOGSTD))


def _edge_body(src_hbm, dst_hbm, hs_hbm, ad_hbm, zeros_hbm, out_hbm,
               src_v, dst_v, rows_v, adr_v, acc, sem1, sem2):
    c = lax.axis_index("c")
    s = lax.axis_index("s")
    wid = s * NC + c

    pltpu.sync_copy(zeros_hbm.at[pl.ds(s * ROWS_PER_TILE, ROWS_PER_TILE)],
                    acc.at[pl.ds(s * ROWS_PER_TILE, ROWS_PER_TILE)])
    plsc.subcore_barrier()

    lane_iota = lax.iota(jnp.int32, 16)

    def chunk_body(i, carry):
        base = (i * NW + wid) * CHUNK
        pltpu.sync_copy(src_hbm.at[pl.ds(base, CHUNK)], src_v)
        pltpu.sync_copy(dst_hbm.at[pl.ds(base, CHUNK)], dst_v)
        cp1 = pltpu.async_copy(hs_hbm.at[src_v], rows_v, sem1)
        cp2 = pltpu.async_copy(ad_hbm.at[dst_v], adr_v, sem2)
        cp1.wait()
        cp2.wait()

        def edge_body(e, carry2):
            av = rows_v[e, pl.ds(D_IN, 16)] + adr_v[e, :]
            ev = jnp.where(av > 0.0, av, av * jnp.float32(0.2))
            exv = jnp.exp(ev)
            rows_v[e, pl.ds(D_IN, 16)] = exv
            for l in range(4):
                b = lax.gather(
                    exv, jnp.full((16, 1), l, jnp.int32),
                    lax.GatherDimensionNumbers(
                        offset_dims=(), collapsed_slice_dims=(0,),
                        start_index_map=(0,)),
                    slice_sizes=(1,),
                    mode=lax.GatherScatterMode.PROMISE_IN_BOUNDS)
                for j in range(2):
                    sl = pl.ds(32 * l + 16 * j, 16)
                    rows_v[e, sl] = rows_v[e, sl] * b
            return carry2

        lax.fori_loop(0, CHUNK, edge_body, 0, unroll=2)
        pltpu.sync_copy(rows_v, acc.at[dst_v], add=True)
        return carry

    lax.fori_loop(0, CHUNKS_PER_TILE, chunk_body, 0)
    plsc.subcore_barrier()
    pltpu.sync_copy(acc.at[pl.ds(s * ROWS_PER_TILE, ROWS_PER_TILE)],
                    out_hbm.at[c, pl.ds(s * ROWS_PER_TILE, ROWS_PER_TILE)])


_edge_kernel = functools.partial(
    pl.kernel,
    out_type=jax.ShapeDtypeStruct((NC, N_PAD, HS_W), jnp.float32),
    mesh=plsc.VectorSubcoreMesh(core_axis_name="c", subcore_axis_name="s"),
    compiler_params=pltpu.CompilerParams(use_tc_tiling_on_sc=False),
    scratch_types=[
        pltpu.VMEM((CHUNK,), jnp.int32),
        pltpu.VMEM((CHUNK,), jnp.int32),
        pltpu.VMEM((CHUNK, HS_W), jnp.float32),
        pltpu.VMEM((CHUNK, AD_W), jnp.float32),
        pltpu.VMEM_SHARED((N_PAD, HS_W), jnp.float32),
        pltpu.SemaphoreType.DMA,
        pltpu.SemaphoreType.DMA,
    ],
)(_edge_body)


def kernel(x, W_mu1, a_s_mu1, a_d_mu1, W_ls1, a_s_ls1, a_d_ls1,
           W_mu2, a_s_mu2, a_d_mu2, W_ls2, a_s_ls2, a_d_ls2, edge_index):
    W_all = jnp.concatenate([W_mu1, W_ls1, W_mu2, W_ls2], axis=1)
    amat_s = jnp.zeros((D_IN, AD_W), jnp.float32)
    amat_d = jnp.zeros((D_IN, AD_W), jnp.float32)
    for l, (a_s, a_d) in enumerate([(a_s_mu1, a_d_mu1), (a_s_ls1, a_d_ls1),
                                    (a_s_mu2, a_d_mu2), (a_s_ls2, a_d_ls2)]):
        amat_s = amat_s.at[32 * l:32 * (l + 1), l].set(a_s)
        amat_d = amat_d.at[32 * l:32 * (l + 1), l].set(a_d)

    hs, ad = pl.pallas_call(
        _proj_body,
        grid=(N // _ROWBLK,),
        in_specs=[
            pl.BlockSpec((_ROWBLK, D_IN), lambda i: (i, 0)),
            pl.BlockSpec((D_IN, D_IN), lambda i: (0, 0)),
            pl.BlockSpec((D_IN, AD_W), lambda i: (0, 0)),
            pl.BlockSpec((D_IN, AD_W), lambda i: (0, 0)),
        ],
        out_specs=[
            pl.BlockSpec((_ROWBLK, HS_W), lambda i: (i, 0)),
            pl.BlockSpec((_ROWBLK, AD_W), lambda i: (i, 0)),
        ],
        out_shape=[
            jax.ShapeDtypeStruct((N, HS_W), jnp.float32),
            jax.ShapeDtypeStruct((N, AD_W), jnp.float32),
        ],
    )(x, W_all, amat_s, amat_d)

    ei = edge_index.astype(jnp.int32)
    src = ei[0]
    dst = ei[1]
    zeros = jnp.zeros((N_PAD, HS_W), jnp.float32)

    partials = _edge_kernel(src, dst, hs, ad, zeros)

    kz = jax.random.split(jax.random.key(42), 2)
    n2 = jax.random.normal(kz[0], (N, LAT), jnp.float32)
    n1 = jax.random.normal(kz[1], (N, LAT), jnp.float32)
    noise = jnp.concatenate([n1, n2], axis=1)

    noise_pad = jnp.zeros((N_PAD, 2 * LAT), jnp.float32).at[:N].set(noise)

    z = pl.pallas_call(
        _finalize_body,
        grid=(N_PAD // _FROWBLK,),
        in_specs=[
            pl.BlockSpec((NC, _FROWBLK, HS_W), lambda i: (0, i, 0)),
            pl.BlockSpec((_FROWBLK, 2 * LAT), lambda i: (i, 0)),
        ],
        out_specs=pl.BlockSpec((_FROWBLK, 2 * LAT), lambda i: (i, 0)),
        out_shape=jax.ShapeDtypeStruct((N_PAD, 2 * LAT), jnp.float32),
    )(partials, noise_pad)
    return z[:N]


# speedup vs baseline: 68.1971x; 1.0718x over previous
import functools

import jax
import jax.numpy as jnp
from jax import lax
from jax.experimental import pallas as pl
from jax.experimental.pallas import tpu as pltpu
from jax.experimental.pallas import tpu_sc as plsc

N = 10000
E = 320000
D_IN = 128
LAT = 32
MAX_LOGSTD = 10.0

HS_W = 144
AD_W = 16
NC = 2
NS = 16
NW = NC * NS
CHUNK = 40
NCHT = 256
E_PAD = NW * NCHT * CHUNK
N_PAD = 10240
RPT = N_PAD // NS

_ROWBLK = 1000
_FROWBLK = 1024


def _proj_body(x_ref, w_ref, amat_s_ref, amat_d_ref, hs_ref, ad_ref):
    h = jnp.dot(x_ref[...], w_ref[...], preferred_element_type=jnp.float32)
    hs_ref[:, 0:D_IN] = h
    hs_ref[:, D_IN:HS_W] = jnp.dot(h, amat_s_ref[...],
                                   preferred_element_type=jnp.float32)
    ad_ref[...] = jnp.dot(h, amat_d_ref[...],
                          preferred_element_type=jnp.float32)


def _finalize_body(p_ref, noise_ref, z_ref):
    a = p_ref[0] + p_ref[1]
    eps = 1e-16
    mu1 = a[:, 0:32] / (a[:, 128:129] + eps)
    ls1 = a[:, 32:64] / (a[:, 129:130] + eps)
    mu2 = a[:, 64:96] / (a[:, 130:131] + eps)
    ls2 = a[:, 96:128] / (a[:, 131:132] + eps)
    z_ref[:, 0:32] = mu1 + noise_ref[:, 0:32] * jnp.exp(
        jnp.minimum(ls1, MAX_LOGSTD))
    z_ref[:, 32:64] = mu2 + noise_ref[:, 32:64] * jnp.exp(
        jnp.minimum(ls2, MAX_LOGSTD))


def _edge_body(pk_hbm, hs_hbm, ad_hbm, zeros_hbm, out_hbm,
               pk, srcv, dstv, rows, adr, acc, isem, gsem, ssem):
    c = lax.axis_index("c")
    s = lax.axis_index("s")
    wid = s * NC + c

    pltpu.async_copy(pk_hbm.at[wid], pk, isem)
    pltpu.sync_copy(zeros_hbm.at[pl.ds(s * RPT, RPT)],
                    acc.at[pl.ds(s * RPT, RPT)])
    pltpu.make_async_copy(pk_hbm.at[wid], pk, isem).wait()
    plsc.subcore_barrier()

    def unpack(b, j):
        for o in (0, 16, 24):
            v = pk[j, pl.ds(o, 16)]
            srcv[b][pl.ds(o, 16)] = lax.shift_right_logical(v, 14)
            dstv[b][pl.ds(o, 16)] = jnp.bitwise_and(v, 16383)

    def fire_gather(b):
        pltpu.async_copy(hs_hbm.at[srcv[b]], rows[b], gsem[b])
        pltpu.async_copy(ad_hbm.at[dstv[b]], adr[b], gsem[b])

    def wait_gather(b):
        pltpu.make_async_copy(hs_hbm.at[srcv[b]], rows[b], gsem[b]).wait()
        pltpu.make_async_copy(ad_hbm.at[dstv[b]], adr[b], gsem[b]).wait()

    def fire_scatter(b):
        pltpu.async_copy(rows[b], acc.at[dstv[b]], ssem[b], add=True)

    def wait_scatter(b):
        pltpu.make_async_copy(rows[b], acc.at[dstv[b]], ssem[b]).wait()

    def compute(b):
        rv = rows[b]
        ar = adr[b]

        def edge_body(e, carry):
            av = rv[e, pl.ds(D_IN, 16)] + ar[e, :]
            ev = jnp.where(av > 0.0, av, av * jnp.float32(0.2))
            exv = jnp.exp(ev)
            rv[e, pl.ds(D_IN, 16)] = exv
            for l in range(4):
                bc = lax.gather(
                    exv, jnp.full((16, 1), l, jnp.int32),
                    lax.GatherDimensionNumbers(
                        offset_dims=(), collapsed_slice_dims=(0,),
                        start_index_map=(0,)),
                    slice_sizes=(1,),
                    mode=lax.GatherScatterMode.PROMISE_IN_BOUNDS)
                for jj in range(2):
                    sl = pl.ds(32 * l + 16 * jj, 16)
                    rv[e, sl] = rv[e, sl] * bc
            return carry

        lax.fori_loop(0, CHUNK, edge_body, 0, unroll=2)

    for b in (0, 1):
        unpack(b, b)
        fire_gather(b)
    for i in (0, 1):
        wait_gather(i)
        compute(i)
        fire_scatter(i)
        unpack(i + 2, i + 2)
        fire_gather(i + 2)

    def macro(g, carry):
        for k in range(4):
            i = 4 * g + 2 + k
            bi = (2 + k) % 4
            b2 = k
            wait_gather(bi)
            compute(bi)
            fire_scatter(bi)
            wait_scatter(b2)
            unpack(b2, i + 2)
            fire_gather(b2)
        return carry

    lax.fori_loop(0, (NCHT - 4) // 4, macro, 0)

    for b in (2, 3):
        wait_gather(b)
        compute(b)
        fire_scatter(b)
    for b in range(4):
        wait_scatter(b)

    plsc.subcore_barrier()
    pltpu.sync_copy(acc.at[pl.ds(s * RPT, RPT)],
                    out_hbm.at[c, pl.ds(s * RPT, RPT)])


_edge_kernel = functools.partial(
    pl.kernel,
    out_type=jax.ShapeDtypeStruct((NC, N_PAD, HS_W), jnp.float32),
    mesh=plsc.VectorSubcoreMesh(core_axis_name="c", subcore_axis_name="s"),
    compiler_params=pltpu.CompilerParams(use_tc_tiling_on_sc=False),
    scratch_types=[
        pltpu.VMEM((NCHT, CHUNK), jnp.int32),
        [pltpu.VMEM((CHUNK,), jnp.int32) for _ in range(4)],
        [pltpu.VMEM((CHUNK,), jnp.int32) for _ in range(4)],
        [pltpu.VMEM((CHUNK, HS_W), jnp.float32) for _ in range(4)],
        [pltpu.VMEM((CHUNK, AD_W), jnp.float32) for _ in range(4)],
        pltpu.VMEM_SHARED((N_PAD, HS_W), jnp.float32),
        pltpu.SemaphoreType.DMA,
        [pltpu.SemaphoreType.DMA for _ in range(4)],
        [pltpu.SemaphoreType.DMA for _ in range(4)],
    ],
)(_edge_body)


def kernel(x, W_mu1, a_s_mu1, a_d_mu1, W_ls1, a_s_ls1, a_d_ls1,
           W_mu2, a_s_mu2, a_d_mu2, W_ls2, a_s_ls2, a_d_ls2, edge_index):
    W_all = jnp.concatenate([W_mu1, W_ls1, W_mu2, W_ls2], axis=1)
    amat_s = jnp.zeros((D_IN, AD_W), jnp.float32)
    amat_d = jnp.zeros((D_IN, AD_W), jnp.float32)
    for l, (a_s, a_d) in enumerate([(a_s_mu1, a_d_mu1), (a_s_ls1, a_d_ls1),
                                    (a_s_mu2, a_d_mu2), (a_s_ls2, a_d_ls2)]):
        amat_s = amat_s.at[32 * l:32 * (l + 1), l].set(a_s)
        amat_d = amat_d.at[32 * l:32 * (l + 1), l].set(a_d)

    hs, ad = pl.pallas_call(
        _proj_body,
        grid=(N // _ROWBLK,),
        in_specs=[
            pl.BlockSpec((_ROWBLK, D_IN), lambda i: (i, 0)),
            pl.BlockSpec((D_IN, D_IN), lambda i: (0, 0)),
            pl.BlockSpec((D_IN, AD_W), lambda i: (0, 0)),
            pl.BlockSpec((D_IN, AD_W), lambda i: (0, 0)),
        ],
        out_specs=[
            pl.BlockSpec((_ROWBLK, HS_W), lambda i: (i, 0)),
            pl.BlockSpec((_ROWBLK, AD_W), lambda i: (i, 0)),
        ],
        out_shape=[
            jax.ShapeDtypeStruct((N, HS_W), jnp.float32),
            jax.ShapeDtypeStruct((N, AD_W), jnp.float32),
        ],
    )(x, W_all, amat_s, amat_d)

    ei = edge_index.astype(jnp.int32)
    pad = E_PAD - E
    src = jnp.concatenate([ei[0], jnp.zeros((pad,), jnp.int32)])
    dst = jnp.concatenate([ei[1], jnp.full((pad,), N_PAD - 1, jnp.int32)])
    packed = (jnp.left_shift(src, 14) | dst).reshape(NW, NCHT, CHUNK)
    zeros = jnp.zeros((N_PAD, HS_W), jnp.float32)

    partials = _edge_kernel(packed, hs, ad, zeros)

    kz = jax.random.split(jax.random.key(42), 2)
    n2 = jax.random.normal(kz[0], (N, LAT), jnp.float32)
    n1 = jax.random.normal(kz[1], (N, LAT), jnp.float32)
    noise = jnp.concatenate([n1, n2], axis=1)
    noise_pad = jnp.zeros((N_PAD, 2 * LAT), jnp.float32).at[:N].set(noise)

    z = pl.pallas_call(
        _finalize_body,
        grid=(N_PAD // _FROWBLK,),
        in_specs=[
            pl.BlockSpec((NC, _FROWBLK, HS_W), lambda i: (0, i, 0)),
            pl.BlockSpec((_FROWBLK, 2 * LAT), lambda i: (i, 0)),
        ],
        out_specs=pl.BlockSpec((_FROWBLK, 2 * LAT), lambda i: (i, 0)),
        out_shape=jax.ShapeDtypeStruct((N_PAD, 2 * LAT), jnp.float32),
    )(partials, noise_pad)
    return z[:N]
